# Initial kernel scaffold; baseline (speedup 1.0000x reference)
import jax
import jax.numpy as jnp
from jax.experimental import pallas as pl


def kernel(x, edge_index, actual_amount, W, b):
    raise NotImplementedError("write your pallas kernel here")


# speedup vs baseline: 11.5562x; 11.5562x over previous
import functools

import jax
import jax.numpy as jnp
from jax import lax
from jax.experimental import pallas as pl
from jax.experimental.pallas import tpu as pltpu
from jax.experimental.pallas import tpu_sc as plsc

N_NODES = 10000
NODE_DIM = 128
PER_GRAPH = 200
NUM_SEG = 50
GROUP = 21
N_EDGES = 210000

NC, NS = 2, 16
NW = NC * NS
GPT = 320
NBLK = GPT // 16
EPT = GPT * GROUP
GPAD = NW * GPT
EPAD = GPAD * GROUP
SEGC = NUM_SEG * 2
PARTW = 112
KEEP = 5


def _mm_body(w_ref, x_ref, b_ref, o_ref):
    o_ref[...] = lax.dot_general(
        w_ref[...], x_ref[...],
        (((1,), (1,)), ((), ())),
        preferred_element_type=jnp.float32,
    ) + b_ref[:, :1]


def _node_table(wpad, x, bmat):
    return pl.pallas_call(
        _mm_body,
        out_shape=jax.ShapeDtypeStruct((8, N_NODES), jnp.float32),
    )(wpad, x, bmat)


_MESH = plsc.VectorSubcoreMesh(
    core_axis_name="c", subcore_axis_name="s", num_cores=NC, num_subcores=NS)
_SC_PARAMS = pltpu.CompilerParams(needs_layout_passes=False)


@functools.partial(
    pl.kernel,
    out_type=(
        jax.ShapeDtypeStruct((EPAD,), jnp.float32),
        jax.ShapeDtypeStruct((EPAD,), jnp.float32),
        jax.ShapeDtypeStruct((NW, SEGC, 16), jnp.float32),
    ),
    mesh=_MESH,
    compiler_params=_SC_PARAMS,
    scratch_types=[
        pltpu.VMEM((4 * N_NODES,), jnp.float32),
        pltpu.VMEM((EPT,), jnp.int32),
        pltpu.VMEM((EPT,), jnp.int32),
        pltpu.VMEM((EPT,), jnp.float32),
        pltpu.VMEM((EPT,), jnp.float32),
        pltpu.VMEM((SEGC, 16), jnp.float32),
    ],
)
def _edge_kernel(t_hbm, row_hbm, col_hbm, att0_hbm, att1_hbm, part_hbm,
                 t_v, row_v, col_v, a0_v, a1_v, acc_v):
    w = lax.axis_index("s") * NC + lax.axis_index("c")
    ebase = w * EPT
    pltpu.sync_copy(t_hbm.at[pl.ds(0, 4 * N_NODES)], t_v)
    pltpu.sync_copy(row_hbm.at[pl.ds(ebase, EPT)], row_v)
    pltpu.sync_copy(col_hbm.at[pl.ds(ebase, EPT)], col_v)

    zero16 = jnp.zeros((16,), jnp.float32)
    for s in range(SEGC):
        acc_v[s] = zero16

    iota = lax.iota(jnp.int32, 16)
    gbase = w * GPT

    def block(bi, carry):
        gl = bi * 16 + iota
        valid = (gbase + gl) < (N_EDGES // GROUP)
        eo = gl * GROUP

        for p in range(GROUP):
            idx = eo + p
            r = plsc.load_gather(row_v, [idx])
            c = plsc.load_gather(col_v, [idx])
            z0 = (plsc.load_gather(t_v, [r])
                  + plsc.load_gather(t_v, [c + 2 * N_NODES]))
            z1 = (plsc.load_gather(t_v, [r + N_NODES])
                  + plsc.load_gather(t_v, [c + 3 * N_NODES]))
            plsc.store_scatter(a0_v, [idx], 1.0 / (1.0 + jnp.exp(-z0)))
            plsc.store_scatter(a1_v, [idx], 1.0 / (1.0 + jnp.exp(-z1)))

        for attv, segoff in ((a0_v, 0), (a1_v, NUM_SEG)):
            vs = [plsc.load_gather(attv, [eo + i]) for i in range(GROUP)]
            beats = [jnp.zeros((16,), jnp.int32) for _ in range(GROUP)]
            for i in range(GROUP):
                for j in range(i + 1, GROUP):
                    ge = (vs[j] >= vs[i]).astype(jnp.int32)
                    beats[i] = beats[i] + ge
                    beats[j] = beats[j] + 1 - ge
            for i in range(GROUP):
                m = jnp.where(beats[i] < KEEP, vs[i], 0.0)
                plsc.store_scatter(attv, [eo + i], m)
                r = plsc.load_gather(row_v, [eo + i])
                seg = lax.div(r, PER_GRAPH) + segoff
                plsc.addupdate_scatter(acc_v, [seg, iota], m, mask=valid)
        return carry

    lax.fori_loop(0, NBLK, block, 0)

    pltpu.sync_copy(acc_v, part_hbm.at[w])
    pltpu.sync_copy(a0_v, att0_hbm.at[pl.ds(ebase, EPT)])
    pltpu.sync_copy(a1_v, att1_hbm.at[pl.ds(ebase, EPT)])


@functools.partial(
    pl.kernel,
    out_type=jax.ShapeDtypeStruct((EPAD,), jnp.float32),
    mesh=_MESH,
    compiler_params=_SC_PARAMS,
    scratch_types=[
        pltpu.VMEM((NW * SEGC * 16,), jnp.float32),
        pltpu.VMEM((SEGC * 16,), jnp.float32),
        pltpu.VMEM((EPT,), jnp.int32),
        pltpu.VMEM((EPT,), jnp.float32),
        pltpu.VMEM((EPT,), jnp.float32),
        pltpu.VMEM((EPT,), jnp.float32),
        pltpu.VMEM((EPT,), jnp.float32),
        pltpu.VMEM((EPT,), jnp.float32),
    ],
)
def _norm_kernel(part_hbm, row_hbm, att0_hbm, att1_hbm, amt0_hbm, amt1_hbm,
                 out_hbm, part_v, inv_v, row_v, a0_v, a1_v, m0_v, m1_v, out_v):
    w = lax.axis_index("s") * NC + lax.axis_index("c")
    ebase = w * EPT
    pltpu.sync_copy(part_hbm, part_v)
    pltpu.sync_copy(row_hbm.at[pl.ds(ebase, EPT)], row_v)
    pltpu.sync_copy(att0_hbm.at[pl.ds(ebase, EPT)], a0_v)
    pltpu.sync_copy(att1_hbm.at[pl.ds(ebase, EPT)], a1_v)
    pltpu.sync_copy(amt0_hbm.at[pl.ds(ebase, EPT)], m0_v)
    pltpu.sync_copy(amt1_hbm.at[pl.ds(ebase, EPT)], m1_v)

    iota = lax.iota(jnp.int32, 16)

    def seg_reduce(s, carry):
        tot = jnp.zeros((16,), jnp.float32)
        for w2 in range(NW):
            tot = tot + plsc.load_gather(part_v, [w2 * SEGC * 16 + s * 16 + iota])
        bs = jnp.broadcast_to(jnp.sum(tot), (16,))
        plsc.store_scatter(inv_v, [s * 16 + iota],
                           1.0 / jnp.maximum(bs, 1.0))
        return carry

    lax.fori_loop(0, SEGC, seg_reduce, 0)

    def eblk(k, carry):
        off = k * 16
        idx = off + iota
        a0 = plsc.load_gather(a0_v, [idx])
        a1 = plsc.load_gather(a1_v, [idx])
        m0 = plsc.load_gather(m0_v, [idx])
        m1 = plsc.load_gather(m1_v, [idx])
        r = plsc.load_gather(row_v, [idx])
        seg = lax.div(r, PER_GRAPH)
        inv0 = plsc.load_gather(inv_v, [seg * 16 + iota])
        inv1 = plsc.load_gather(inv_v, [(seg + NUM_SEG) * 16 + iota])
        plsc.store_scatter(out_v, [idx], a0 * m0 * inv0 + a1 * m1 * inv1)
        return carry

    lax.fori_loop(0, EPT // 16, eblk, 0)
    pltpu.sync_copy(out_v, out_hbm.at[pl.ds(ebase, EPT)])


def kernel(x, edge_index, actual_amount, W, b):
    wfull = jnp.concatenate([W[:, :NODE_DIM], W[:, NODE_DIM:]], axis=0)
    wpad = jnp.zeros((8, NODE_DIM), jnp.float32).at[:4].set(wfull)
    b4 = jnp.zeros((8,), jnp.float32).at[:2].set(b)
    bmat = jnp.tile(b4[:, None], (1, NODE_DIM))
    table = _node_table(wpad, x, bmat).reshape(-1)

    pad = EPAD - N_EDGES
    rowp = jnp.pad(edge_index[0], (0, pad))
    colp = jnp.pad(edge_index[1], (0, pad))
    amt0 = jnp.pad(actual_amount[:, 0], (0, pad))
    amt1 = jnp.pad(actual_amount[:, 1], (0, pad))

    att0, att1, part = _edge_kernel(table, rowp, colp)
    out = _norm_kernel(part.reshape(-1), rowp, att0, att1, amt0, amt1)
    return out[:N_EDGES]


# speedup vs baseline: 13.5341x; 1.1712x over previous
import functools

import jax
import jax.numpy as jnp
from jax import lax
from jax.experimental import pallas as pl
from jax.experimental.pallas import tpu as pltpu
from jax.experimental.pallas import tpu_sc as plsc

N_NODES = 10000
NODE_DIM = 128
PER_GRAPH = 200
NUM_SEG = 50
GROUP = 21
N_EDGES = 210000
NGROUPS = N_EDGES // GROUP

NC, NS = 2, 16
NW = NC * NS
GPT = 320
NBLK = GPT // 16
EPT = GPT * GROUP
EPAD = NW * EPT
SEGC = NUM_SEG * 2
SEGP = 112
KEEP = 5
UNROLL = 3

CE5 = ((0, 1), (3, 4), (2, 4), (2, 3), (0, 3), (0, 2), (1, 4), (1, 3), (1, 2))


def _mm_body(w_ref, x_ref, b_ref, o_ref):
    o_ref[...] = lax.dot_general(
        w_ref[...], x_ref[...],
        (((1,), (1,)), ((), ())),
        preferred_element_type=jnp.float32,
    ) + b_ref[:, :1]


def _node_table(wpad, x, bmat):
    return pl.pallas_call(
        _mm_body,
        out_shape=jax.ShapeDtypeStruct((8, N_NODES), jnp.float32),
    )(wpad, x, bmat)


_MESH = plsc.VectorSubcoreMesh(
    core_axis_name="c", subcore_axis_name="s", num_cores=NC, num_subcores=NS)
_SC_PARAMS = pltpu.CompilerParams(needs_layout_passes=False)


@functools.partial(
    pl.kernel,
    out_type=(
        jax.ShapeDtypeStruct((EPAD,), jnp.float32),
        jax.ShapeDtypeStruct((EPAD,), jnp.float32),
        jax.ShapeDtypeStruct((NW, SEGP, 16), jnp.float32),
    ),
    mesh=_MESH,
    compiler_params=_SC_PARAMS,
    scratch_types=[
        pltpu.VMEM((4 * N_NODES,), jnp.float32),
        pltpu.VMEM((EPT,), jnp.int32),
        pltpu.VMEM((EPT,), jnp.int32),
        pltpu.VMEM((EPT,), jnp.float32),
        pltpu.VMEM((EPT,), jnp.float32),
        pltpu.VMEM((SEGP, 16), jnp.float32),
    ],
)
def _edge_kernel(t_hbm, row_hbm, col_hbm, att0_hbm, att1_hbm, part_hbm,
                 t_v, row_v, col_v, a0_v, a1_v, acc_v):
    w = lax.axis_index("s") * NC + lax.axis_index("c")
    ebase = w * EPT
    pltpu.sync_copy(t_hbm.at[pl.ds(0, 4 * N_NODES)], t_v)
    pltpu.sync_copy(row_hbm.at[pl.ds(ebase, EPT)], row_v)
    pltpu.sync_copy(col_hbm.at[pl.ds(ebase, EPT)], col_v)

    zero16 = jnp.zeros((16,), jnp.float32)
    for s in range(SEGP):
        acc_v[s] = zero16

    iota = lax.iota(jnp.int32, 16)
    gbase = w * GPT

    def p1(k, carry):
        for u in range(UNROLL):
            idx = (k * UNROLL + u) * 16 + iota
            r = plsc.load_gather(row_v, [idx])
            c = plsc.load_gather(col_v, [idx])
            z0 = (plsc.load_gather(t_v, [r])
                  + plsc.load_gather(t_v, [c + 2 * N_NODES]))
            z1 = (plsc.load_gather(t_v, [r + N_NODES])
                  + plsc.load_gather(t_v, [c + 3 * N_NODES]))
            plsc.store_scatter(a0_v, [idx], 1.0 / (1.0 + jnp.exp(-z0)))
            plsc.store_scatter(a1_v, [idx], 1.0 / (1.0 + jnp.exp(-z1)))
        return carry

    lax.fori_loop(0, EPT // (16 * UNROLL), p1, 0)

    def block(bi, carry):
        gl = bi * 16 + iota
        valid = (gbase + gl) < NGROUPS
        eo = gl * GROUP
        for attv, segoff in ((a0_v, 0), (a1_v, NUM_SEG)):
            vs = [plsc.load_gather(attv, [eo + i]) for i in range(GROUP)]
            s = list(vs[:KEEP])
            for a, b in CE5:
                hi = jnp.maximum(s[a], s[b])
                s[b] = jnp.minimum(s[a], s[b])
                s[a] = hi
            for i in range(KEEP, GROUP):
                x = vs[i]
                for k2 in range(KEEP - 1):
                    lo = jnp.minimum(s[k2], x)
                    s[k2] = jnp.maximum(s[k2], x)
                    x = lo
                s[KEEP - 1] = jnp.maximum(s[KEEP - 1], x)
            t = s[KEEP - 1]
            cnt = jnp.zeros((16,), jnp.int32)
            for i in range(GROUP):
                cnt = cnt + (vs[i] > t).astype(jnp.int32)
            allowed = KEEP - cnt
            acc = jnp.zeros((16,), jnp.int32)
            for i in range(GROUP - 1, -1, -1):
                eq = vs[i] == t
                kp = (vs[i] > t) | (eq & (acc < allowed))
                m = jnp.where(kp, vs[i], 0.0)
                plsc.store_scatter(attv, [eo + i], m)
                r = plsc.load_gather(row_v, [eo + i])
                seg = lax.div(r, PER_GRAPH) + segoff
                plsc.addupdate_scatter(acc_v, [seg, iota], m, mask=valid)
                acc = acc + eq.astype(jnp.int32)
        return carry

    lax.fori_loop(0, NBLK, block, 0)

    for s in range(SEGC):
        acc_v[s] = jnp.broadcast_to(jnp.sum(acc_v[s]), (16,))

    pltpu.sync_copy(acc_v, part_hbm.at[w])
    pltpu.sync_copy(a0_v, att0_hbm.at[pl.ds(ebase, EPT)])
    pltpu.sync_copy(a1_v, att1_hbm.at[pl.ds(ebase, EPT)])


@functools.partial(
    pl.kernel,
    out_type=jax.ShapeDtypeStruct((EPAD,), jnp.float32),
    mesh=_MESH,
    compiler_params=_SC_PARAMS,
    scratch_types=[
        pltpu.VMEM((NW * SEGP * 16,), jnp.float32),
        pltpu.VMEM((SEGP,), jnp.float32),
        pltpu.VMEM((EPT,), jnp.int32),
        pltpu.VMEM((EPT,), jnp.float32),
        pltpu.VMEM((EPT,), jnp.float32),
        pltpu.VMEM((EPT,), jnp.float32),
        pltpu.VMEM((EPT,), jnp.float32),
        pltpu.VMEM((EPT,), jnp.float32),
    ],
)
def _norm_kernel(part_hbm, row_hbm, att0_hbm, att1_hbm, amt0_hbm, amt1_hbm,
                 out_hbm, part_v, inv_v, row_v, a0_v, a1_v, m0_v, m1_v, out_v):
    w = lax.axis_index("s") * NC + lax.axis_index("c")
    ebase = w * EPT
    pltpu.sync_copy(part_hbm, part_v)
    pltpu.sync_copy(row_hbm.at[pl.ds(ebase, EPT)], row_v)
    pltpu.sync_copy(att0_hbm.at[pl.ds(ebase, EPT)], a0_v)
    pltpu.sync_copy(att1_hbm.at[pl.ds(ebase, EPT)], a1_v)
    pltpu.sync_copy(amt0_hbm.at[pl.ds(ebase, EPT)], m0_v)
    pltpu.sync_copy(amt1_hbm.at[pl.ds(ebase, EPT)], m1_v)

    iota = lax.iota(jnp.int32, 16)

    for sb in range(SEGP // 16):
        segs = sb * 16 + iota
        tot = jnp.zeros((16,), jnp.float32)
        for w2 in range(NW):
            tot = tot + plsc.load_gather(part_v, [w2 * SEGP * 16 + segs * 16])
        inv_v[pl.ds(sb * 16, 16)] = 1.0 / jnp.maximum(tot, 1.0)

    def eblk(k, carry):
        for u in range(UNROLL):
            idx = (k * UNROLL + u) * 16 + iota
            a0 = plsc.load_gather(a0_v, [idx])
            a1 = plsc.load_gather(a1_v, [idx])
            m0 = plsc.load_gather(m0_v, [idx])
            m1 = plsc.load_gather(m1_v, [idx])
            r = plsc.load_gather(row_v, [idx])
            seg = lax.div(r, PER_GRAPH)
            inv0 = plsc.load_gather(inv_v, [seg])
            inv1 = plsc.load_gather(inv_v, [seg + NUM_SEG])
            plsc.store_scatter(out_v, [idx], a0 * m0 * inv0 + a1 * m1 * inv1)
        return carry

    lax.fori_loop(0, EPT // (16 * UNROLL), eblk, 0)
    pltpu.sync_copy(out_v, out_hbm.at[pl.ds(ebase, EPT)])


def kernel(x, edge_index, actual_amount, W, b):
    wfull = jnp.concatenate([W[:, :NODE_DIM], W[:, NODE_DIM:]], axis=0)
    wpad = jnp.zeros((8, NODE_DIM), jnp.float32).at[:4].set(wfull)
    b4 = jnp.zeros((8,), jnp.float32).at[:2].set(b)
    bmat = jnp.tile(b4[:, None], (1, NODE_DIM))
    table = _node_table(wpad, x, bmat).reshape(-1)

    pad = EPAD - N_EDGES
    rowp = jnp.pad(edge_index[0], (0, pad))
    colp = jnp.pad(edge_index[1], (0, pad))
    amt0 = jnp.pad(actual_amount[:, 0], (0, pad))
    amt1 = jnp.pad(actual_amount[:, 1], (0, pad))

    att0, att1, part = _edge_kernel(table, rowp, colp)
    out = _norm_kernel(part.reshape(-1), rowp, att0, att1, amt0, amt1)
    return out[:N_EDGES]


# speedup vs baseline: 26.8978x; 1.9874x over previous
import functools

import jax
import jax.numpy as jnp
from jax import lax
from jax.experimental import pallas as pl
from jax.experimental.pallas import tpu as pltpu
from jax.experimental.pallas import tpu_sc as plsc

N_NODES = 10000
NODE_DIM = 128
PER_GRAPH = 200
NUM_SEG = 50
GROUP = 21
N_EDGES = 210000
NGROUPS = N_EDGES // GROUP

NC, NS = 2, 16
NW = NC * NS
GPT = 320
NBLK = GPT // 16
EPT = GPT * GROUP
EPAD = NW * EPT
SEGC = NUM_SEG * 2
SEGP = 112
KEEP = 5
UNROLL = 3
INV200 = 0.0050000012

CE5 = ((0, 1), (3, 4), (2, 4), (2, 3), (0, 3), (0, 2), (1, 4), (1, 3), (1, 2))


def _mm_body(w_ref, x_ref, b_ref, o_ref):
    o_ref[...] = lax.dot_general(
        w_ref[...], x_ref[...],
        (((1,), (1,)), ((), ())),
        preferred_element_type=jnp.float32,
    ) + b_ref[:, :1]


def _node_table(wpad, x, bmat):
    return pl.pallas_call(
        _mm_body,
        out_shape=jax.ShapeDtypeStruct((8, N_NODES), jnp.float32),
    )(wpad, x, bmat)


_MESH = plsc.VectorSubcoreMesh(
    core_axis_name="c", subcore_axis_name="s", num_cores=NC, num_subcores=NS)
_SC_PARAMS = pltpu.CompilerParams(needs_layout_passes=False)


@functools.partial(
    pl.kernel,
    out_type=(
        jax.ShapeDtypeStruct((EPAD,), jnp.float32),
        jax.ShapeDtypeStruct((EPAD,), jnp.float32),
        jax.ShapeDtypeStruct((NW, SEGP, 16), jnp.float32),
    ),
    mesh=_MESH,
    compiler_params=_SC_PARAMS,
    scratch_types=[
        pltpu.VMEM((4 * N_NODES,), jnp.float32),
        pltpu.VMEM((EPT,), jnp.int32),
        pltpu.VMEM((EPT,), jnp.int32),
        pltpu.VMEM((EPT,), jnp.float32),
        pltpu.VMEM((EPT,), jnp.float32),
        pltpu.VMEM((EPT,), jnp.int32),
        pltpu.VMEM((SEGP, 16), jnp.float32),
    ],
)
def _edge_kernel(t_hbm, row_hbm, col_hbm, att0_hbm, att1_hbm, part_hbm,
                 t_v, row_v, col_v, a0_v, a1_v, seg_v, acc_v):
    w = lax.axis_index("s") * NC + lax.axis_index("c")
    ebase = w * EPT
    pltpu.sync_copy(t_hbm.at[pl.ds(0, 4 * N_NODES)], t_v)
    pltpu.sync_copy(row_hbm.at[pl.ds(ebase, EPT)], row_v)
    pltpu.sync_copy(col_hbm.at[pl.ds(ebase, EPT)], col_v)

    zero16 = jnp.zeros((16,), jnp.float32)
    for s in range(SEGP):
        acc_v[s] = zero16

    iota = lax.iota(jnp.int32, 16)
    gbase = w * GPT

    def p1(k, carry):
        for u in range(UNROLL):
            idx = (k * UNROLL + u) * 16 + iota
            r = plsc.load_gather(row_v, [idx])
            c = plsc.load_gather(col_v, [idx])
            z0 = (plsc.load_gather(t_v, [r])
                  + plsc.load_gather(t_v, [c + 2 * N_NODES]))
            z1 = (plsc.load_gather(t_v, [r + N_NODES])
                  + plsc.load_gather(t_v, [c + 3 * N_NODES]))
            plsc.store_scatter(a0_v, [idx], 1.0 / (1.0 + jnp.exp(-z0)))
            plsc.store_scatter(a1_v, [idx], 1.0 / (1.0 + jnp.exp(-z1)))
            seg = (r.astype(jnp.float32) * INV200).astype(jnp.int32)
            plsc.store_scatter(seg_v, [idx], seg)
        return carry

    lax.fori_loop(0, EPT // (16 * UNROLL), p1, 0)

    def block(bi, carry):
        gl = bi * 16 + iota
        valid = (gbase + gl) < NGROUPS
        eo = gl * GROUP
        for attv, segoff in ((a0_v, 0), (a1_v, NUM_SEG)):
            vs = [plsc.load_gather(attv, [eo + i]) for i in range(GROUP)]
            s = list(vs[:KEEP])
            for a, b in CE5:
                hi = jnp.maximum(s[a], s[b])
                s[b] = jnp.minimum(s[a], s[b])
                s[a] = hi
            for i in range(KEEP, GROUP):
                x = vs[i]
                for k2 in range(KEEP - 1):
                    lo = jnp.minimum(s[k2], x)
                    s[k2] = jnp.maximum(s[k2], x)
                    x = lo
                s[KEEP - 1] = jnp.maximum(s[KEEP - 1], x)
            t = s[KEEP - 1]
            cnt = jnp.zeros((16,), jnp.int32)
            for i in range(GROUP):
                cnt = cnt + (vs[i] > t).astype(jnp.int32)
            allowed = KEEP - cnt
            acc = jnp.zeros((16,), jnp.int32)
            for i in range(GROUP - 1, -1, -1):
                eq = vs[i] == t
                kp = (vs[i] > t) | (eq & (acc < allowed))
                m = jnp.where(kp, vs[i], 0.0)
                plsc.store_scatter(attv, [eo + i], m)
                seg = plsc.load_gather(seg_v, [eo + i]) + segoff
                plsc.addupdate_scatter(acc_v, [seg, iota], m, mask=valid)
                acc = acc + eq.astype(jnp.int32)
        return carry

    lax.fori_loop(0, NBLK, block, 0)

    for s in range(SEGC):
        acc_v[s] = jnp.broadcast_to(jnp.sum(acc_v[s]), (16,))

    pltpu.sync_copy(acc_v, part_hbm.at[w])
    pltpu.sync_copy(a0_v, att0_hbm.at[pl.ds(ebase, EPT)])
    pltpu.sync_copy(a1_v, att1_hbm.at[pl.ds(ebase, EPT)])


@functools.partial(
    pl.kernel,
    out_type=jax.ShapeDtypeStruct((EPAD,), jnp.float32),
    mesh=_MESH,
    compiler_params=_SC_PARAMS,
    scratch_types=[
        pltpu.VMEM((NW * SEGP * 16,), jnp.float32),
        pltpu.VMEM((SEGP,), jnp.float32),
        pltpu.VMEM((EPT,), jnp.int32),
        pltpu.VMEM((EPT,), jnp.float32),
        pltpu.VMEM((EPT,), jnp.float32),
        pltpu.VMEM((EPT,), jnp.float32),
        pltpu.VMEM((EPT,), jnp.float32),
        pltpu.VMEM((EPT,), jnp.float32),
    ],
)
def _norm_kernel(part_hbm, row_hbm, att0_hbm, att1_hbm, amt0_hbm, amt1_hbm,
                 out_hbm, part_v, inv_v, row_v, a0_v, a1_v, m0_v, m1_v, out_v):
    w = lax.axis_index("s") * NC + lax.axis_index("c")
    ebase = w * EPT
    pltpu.sync_copy(part_hbm, part_v)
    pltpu.sync_copy(row_hbm.at[pl.ds(ebase, EPT)], row_v)
    pltpu.sync_copy(att0_hbm.at[pl.ds(ebase, EPT)], a0_v)
    pltpu.sync_copy(att1_hbm.at[pl.ds(ebase, EPT)], a1_v)
    pltpu.sync_copy(amt0_hbm.at[pl.ds(ebase, EPT)], m0_v)
    pltpu.sync_copy(amt1_hbm.at[pl.ds(ebase, EPT)], m1_v)

    iota = lax.iota(jnp.int32, 16)

    for sb in range(SEGP // 16):
        segs = sb * 16 + iota
        tot = jnp.zeros((16,), jnp.float32)
        for w2 in range(NW):
            tot = tot + plsc.load_gather(part_v, [w2 * SEGP * 16 + segs * 16])
        inv_v[pl.ds(sb * 16, 16)] = 1.0 / jnp.maximum(tot, 1.0)

    def eblk(k, carry):
        for u in range(UNROLL):
            idx = (k * UNROLL + u) * 16 + iota
            a0 = plsc.load_gather(a0_v, [idx])
            a1 = plsc.load_gather(a1_v, [idx])
            m0 = plsc.load_gather(m0_v, [idx])
            m1 = plsc.load_gather(m1_v, [idx])
            r = plsc.load_gather(row_v, [idx])
            seg = (r.astype(jnp.float32) * INV200).astype(jnp.int32)
            inv0 = plsc.load_gather(inv_v, [seg])
            inv1 = plsc.load_gather(inv_v, [seg + NUM_SEG])
            plsc.store_scatter(out_v, [idx], a0 * m0 * inv0 + a1 * m1 * inv1)
        return carry

    lax.fori_loop(0, EPT // (16 * UNROLL), eblk, 0)
    pltpu.sync_copy(out_v, out_hbm.at[pl.ds(ebase, EPT)])


def kernel(x, edge_index, actual_amount, W, b):
    wfull = jnp.concatenate([W[:, :NODE_DIM], W[:, NODE_DIM:]], axis=0)
    wpad = jnp.zeros((8, NODE_DIM), jnp.float32).at[:4].set(wfull)
    b4 = jnp.zeros((8,), jnp.float32).at[:2].set(b)
    bmat = jnp.tile(b4[:, None], (1, NODE_DIM))
    table = _node_table(wpad, x, bmat).reshape(-1)

    pad = EPAD - N_EDGES
    rowp = jnp.pad(edge_index[0], (0, pad))
    colp = jnp.pad(edge_index[1], (0, pad))
    amt0 = jnp.pad(actual_amount[:, 0], (0, pad))
    amt1 = jnp.pad(actual_amount[:, 1], (0, pad))

    att0, att1, part = _edge_kernel(table, rowp, colp)
    out = _norm_kernel(part.reshape(-1), rowp, att0, att1, amt0, amt1)
    return out[:N_EDGES]


# speedup vs baseline: 31.1525x; 1.1582x over previous
import functools

import jax
import jax.numpy as jnp
from jax import lax
from jax.experimental import pallas as pl
from jax.experimental.pallas import tpu as pltpu
from jax.experimental.pallas import tpu_sc as plsc

N_NODES = 10000
NODE_DIM = 128
PER_GRAPH = 200
NUM_SEG = 50
GROUP = 21
N_EDGES = 210000
NGROUPS = N_EDGES // GROUP

NC, NS = 2, 16
NW = NC * NS
GPT = 320
NBLK = GPT // 16
EPT = GPT * GROUP
EPAD = NW * EPT
LAST_N = N_EDGES - (NW - 1) * EPT
SEGC = NUM_SEG * 2
SEGP = 112
KEEP = 5
UNROLL = 3
INV200 = 0.0050000012

CE5 = ((0, 1), (3, 4), (2, 4), (2, 3), (0, 3), (0, 2), (1, 4), (1, 3), (1, 2))


def _mm_body(w_ref, x_ref, b_ref, o_ref):
    o_ref[...] = lax.dot_general(
        w_ref[...], x_ref[...],
        (((1,), (1,)), ((), ())),
        preferred_element_type=jnp.float32,
    ) + b_ref[:, :1]


def _node_table(wpad, x, bmat):
    return pl.pallas_call(
        _mm_body,
        out_shape=jax.ShapeDtypeStruct((8, N_NODES), jnp.float32),
    )(wpad, x, bmat)


_MESH = plsc.VectorSubcoreMesh(
    core_axis_name="c", subcore_axis_name="s", num_cores=NC, num_subcores=NS)
_SC_PARAMS = pltpu.CompilerParams(needs_layout_passes=False)


@functools.partial(
    pl.kernel,
    out_type=(
        jax.ShapeDtypeStruct((EPAD,), jnp.float32),
        jax.ShapeDtypeStruct((EPAD,), jnp.float32),
        jax.ShapeDtypeStruct((NW, SEGP), jnp.float32),
    ),
    mesh=_MESH,
    compiler_params=_SC_PARAMS,
    scratch_types=[
        pltpu.VMEM((4 * N_NODES,), jnp.float32),
        pltpu.VMEM((EPT,), jnp.int32),
        pltpu.VMEM((EPT,), jnp.int32),
        pltpu.VMEM((EPT,), jnp.float32),
        pltpu.VMEM((EPT,), jnp.float32),
        pltpu.VMEM((EPT,), jnp.int32),
        pltpu.VMEM((SEGP, 16), jnp.float32),
        pltpu.VMEM((SEGP,), jnp.float32),
        pltpu.SemaphoreType.DMA,
    ],
)
def _edge_kernel(t_hbm, roww_hbm, colw_hbm, att0_hbm, att1_hbm, part_hbm,
                 t_v, row_v, col_v, a0_v, a1_v, seg_v, acc_v, cp_v, sem):
    w = lax.axis_index("s") * NC + lax.axis_index("c")
    ebase = w * EPT
    c1 = pltpu.async_copy(t_hbm.at[pl.ds(0, 4 * N_NODES)], t_v, sem)
    c2 = pltpu.async_copy(roww_hbm.at[pl.ds(ebase, EPT)], row_v, sem)
    c3 = pltpu.async_copy(colw_hbm.at[pl.ds(ebase, EPT)], col_v, sem)

    zero16 = jnp.zeros((16,), jnp.float32)
    for s in range(SEGP):
        acc_v[s] = zero16

    iota = lax.iota(jnp.int32, 16)
    gbase = w * GPT
    c1.wait()
    c2.wait()
    c3.wait()

    def p1(k, carry):
        for u in range(UNROLL):
            idx = (k * UNROLL + u) * 16 + iota
            r = plsc.load_gather(row_v, [idx])
            c = plsc.load_gather(col_v, [idx])
            z0 = (plsc.load_gather(t_v, [r])
                  + plsc.load_gather(t_v, [c + 2 * N_NODES]))
            z1 = (plsc.load_gather(t_v, [r + N_NODES])
                  + plsc.load_gather(t_v, [c + 3 * N_NODES]))
            plsc.store_scatter(a0_v, [idx], 1.0 / (1.0 + jnp.exp(-z0)))
            plsc.store_scatter(a1_v, [idx], 1.0 / (1.0 + jnp.exp(-z1)))
            seg = (r.astype(jnp.float32) * INV200).astype(jnp.int32)
            plsc.store_scatter(seg_v, [idx], seg)
        return carry

    lax.fori_loop(0, EPT // (16 * UNROLL), p1, 0)

    def block(bi, carry):
        gl = bi * 16 + iota
        valid = (gbase + gl) < NGROUPS
        eo = gl * GROUP
        for attv, segoff in ((a0_v, 0), (a1_v, NUM_SEG)):
            vs = [plsc.load_gather(attv, [eo + i]) for i in range(GROUP)]
            s = list(vs[:KEEP])
            for a, b in CE5:
                hi = jnp.maximum(s[a], s[b])
                s[b] = jnp.minimum(s[a], s[b])
                s[a] = hi
            for i in range(KEEP, GROUP):
                x = vs[i]
                for k2 in range(KEEP - 1):
                    lo = jnp.minimum(s[k2], x)
                    s[k2] = jnp.maximum(s[k2], x)
                    x = lo
                s[KEEP - 1] = jnp.maximum(s[KEEP - 1], x)
            t = s[KEEP - 1]
            cnt = jnp.zeros((16,), jnp.int32)
            for i in range(GROUP):
                cnt = cnt + (vs[i] > t).astype(jnp.int32)
            allowed = KEEP - cnt
            acc = jnp.zeros((16,), jnp.int32)
            for i in range(GROUP - 1, -1, -1):
                eq = vs[i] == t
                kp = (vs[i] > t) | (eq & (acc < allowed))
                m = jnp.where(kp, vs[i], 0.0)
                plsc.store_scatter(attv, [eo + i], m)
                seg = plsc.load_gather(seg_v, [eo + i]) + segoff
                plsc.addupdate_scatter(acc_v, [seg, iota], m, mask=valid)
                acc = acc + eq.astype(jnp.int32)
        return carry

    lax.fori_loop(0, NBLK, block, 0)

    zp = jnp.zeros((16,), jnp.float32)
    for sb in range(SEGP // 16):
        cp_v[pl.ds(sb * 16, 16)] = zp
    for s in range(SEGC):
        tot = jnp.broadcast_to(jnp.sum(acc_v[s]), (16,))
        plsc.store_scatter(cp_v, [jnp.broadcast_to(s, (16,))], tot,
                           mask=iota == (s % 16))

    pltpu.sync_copy(cp_v, part_hbm.at[w])
    pltpu.sync_copy(a0_v, att0_hbm.at[pl.ds(ebase, EPT)])
    pltpu.sync_copy(a1_v, att1_hbm.at[pl.ds(ebase, EPT)])


@functools.partial(
    pl.kernel,
    out_type=jax.ShapeDtypeStruct((N_EDGES,), jnp.float32),
    mesh=_MESH,
    compiler_params=_SC_PARAMS,
    scratch_types=[
        pltpu.VMEM((NW * SEGP,), jnp.float32),
        pltpu.VMEM((SEGP,), jnp.float32),
        pltpu.VMEM((EPT,), jnp.int32),
        pltpu.VMEM((EPT,), jnp.float32),
        pltpu.VMEM((EPT,), jnp.float32),
        pltpu.VMEM((EPT,), jnp.float32),
        pltpu.VMEM((EPT,), jnp.float32),
        pltpu.VMEM((EPT,), jnp.float32),
        pltpu.SemaphoreType.DMA,
    ],
)
def _norm_kernel(part_hbm, roww_hbm, att0_hbm, att1_hbm, amt0_hbm, amt1_hbm,
                 out_hbm,
                 part_v, inv_v, row_v, a0_v, a1_v, m0_v, m1_v, out_v, sem):
    w = lax.axis_index("s") * NC + lax.axis_index("c")
    ebase = w * EPT
    c1 = pltpu.async_copy(part_hbm, part_v, sem)
    c2 = pltpu.async_copy(roww_hbm.at[pl.ds(ebase, EPT)], row_v, sem)
    c3 = pltpu.async_copy(att0_hbm.at[pl.ds(ebase, EPT)], a0_v, sem)
    c4 = pltpu.async_copy(att1_hbm.at[pl.ds(ebase, EPT)], a1_v, sem)
    c5 = pltpu.async_copy(amt0_hbm.at[pl.ds(ebase, EPT)], m0_v, sem)
    c6 = pltpu.async_copy(amt1_hbm.at[pl.ds(ebase, EPT)], m1_v, sem)

    iota = lax.iota(jnp.int32, 16)

    c1.wait()
    for sb in range(SEGP // 16):
        segs = sb * 16 + iota
        tot = jnp.zeros((16,), jnp.float32)
        for w2 in range(NW):
            tot = tot + plsc.load_gather(part_v, [w2 * SEGP + segs])
        inv_v[pl.ds(sb * 16, 16)] = 1.0 / jnp.maximum(tot, 1.0)

    c2.wait()
    c3.wait()
    c4.wait()
    c5.wait()
    c6.wait()

    def eblk(k, carry):
        for u in range(UNROLL):
            idx = (k * UNROLL + u) * 16 + iota
            a0 = plsc.load_gather(a0_v, [idx])
            a1 = plsc.load_gather(a1_v, [idx])
            m0 = plsc.load_gather(m0_v, [idx])
            m1 = plsc.load_gather(m1_v, [idx])
            r = plsc.load_gather(row_v, [idx])
            seg = (r.astype(jnp.float32) * INV200).astype(jnp.int32)
            inv0 = plsc.load_gather(inv_v, [seg])
            inv1 = plsc.load_gather(inv_v, [seg + NUM_SEG])
            plsc.store_scatter(out_v, [idx], a0 * m0 * inv0 + a1 * m1 * inv1)
        return carry

    lax.fori_loop(0, EPT // (16 * UNROLL), eblk, 0)

    @pl.when(w < NW - 1)
    def _store_full():
        pltpu.sync_copy(out_v, out_hbm.at[pl.ds(ebase, EPT)])

    @pl.when(w == NW - 1)
    def _store_tail():
        pltpu.sync_copy(out_v.at[pl.ds(0, LAST_N)],
                        out_hbm.at[pl.ds(ebase, LAST_N)])


def kernel(x, edge_index, actual_amount, W, b):
    wfull = jnp.concatenate([W[:, :NODE_DIM], W[:, NODE_DIM:]], axis=0)
    wpad = jnp.zeros((8, NODE_DIM), jnp.float32).at[:4].set(wfull)
    b4 = jnp.zeros((8,), jnp.float32).at[:2].set(b)
    bmat = jnp.tile(b4[:, None], (1, NODE_DIM))
    table = _node_table(wpad, x, bmat).reshape(-1)

    pad = EPAD - N_EDGES
    row_p = jnp.pad(edge_index[0], (0, pad))
    col_p = jnp.pad(edge_index[1], (0, pad))
    amt0_p = jnp.pad(actual_amount[:, 0], (0, pad))
    amt1_p = jnp.pad(actual_amount[:, 1], (0, pad))

    att0, att1, part = _edge_kernel(table, row_p, col_p)
    return _norm_kernel(part.reshape(-1), row_p, att0, att1, amt0_p, amt1_p)
